# baseline (speedup 1.0000x reference)
import functools

import jax
import jax.numpy as jnp
from jax import lax
from jax.experimental import pallas as pl
from jax.experimental.pallas import tpu as pltpu
from jax.experimental.pallas import tpu_sc as plsc

_N = 10000
_NP = 10240
_E = 320000
_D = 128
_TOPKS = (8000, 6400, 5120)

_info = plsc.get_sparse_core_info()
_NC = _info.num_cores
_NS = _info.num_subcores
_NW = _NC * _NS
_EW = _E // _NW
_CH = 40
_NCH = _EW // _CH
_NB = 5
_ND = 2 * _NB
_RPS = _NP // _NS

_mesh = plsc.VectorSubcoreMesh(core_axis_name="c", subcore_axis_name="s")


def _zero_vec16(ref, n):
    z = jnp.zeros((16,), jnp.float32)
    for t in range(n // 16):
        ref[pl.ds(t * 16, 16)] = z


@functools.partial(
    pl.kernel,
    mesh=_mesh,
    out_type=jax.ShapeDtypeStruct((_NP, _D), jnp.float32),
    scratch_types=[
        pltpu.VMEM((_CH,), jnp.int32),
        pltpu.VMEM((_CH, _D), jnp.float32),
        pltpu.SemaphoreType.DMA,
    ],
)
def _emb_gather(xpad_hbm, emb_hbm, out_hbm, idx_v, rows_v, sem):
    c = lax.axis_index("c")
    s = lax.axis_index("s")
    w = c * _NS + s
    rows_per_w = _NP // _NW
    for j in range(rows_per_w // _CH):
        off = w * rows_per_w + j * _CH
        pltpu.sync_copy(xpad_hbm.at[pl.ds(off, _CH)], idx_v)
        pltpu.async_copy(emb_hbm.at[idx_v], rows_v, sem).wait()
        pltpu.sync_copy(rows_v, out_hbm.at[pl.ds(off, _CH)])


@functools.partial(
    pl.kernel,
    mesh=_mesh,
    out_type=[
        jax.ShapeDtypeStruct((_NC, _NP, _D), jnp.float32),
        jax.ShapeDtypeStruct((_NC, _NP), jnp.float32),
    ],
    scratch_types=[
        pltpu.VMEM_SHARED((_NP, _D), jnp.float32),
        pltpu.VMEM_SHARED((_NP,), jnp.float32),
    ] + [pltpu.VMEM((_CH, _D), jnp.float32) for _ in range(_NB)]
      + [pltpu.VMEM((_CH,), jnp.float32) for _ in range(_NB)]
      + [pltpu.VMEM((_CH,), jnp.int32) for _ in range(_NB)]
      + [pltpu.VMEM((_CH,), jnp.int32) for _ in range(_NB)]
      + [
        pltpu.VMEM((8, _D), jnp.float32),
        pltpu.VMEM((_RPS,), jnp.float32),
        pltpu.SemaphoreType.DMA((_NB,)),
        pltpu.SemaphoreType.DMA((_NB,)),
        pltpu.SemaphoreType.DMA((_NB,)),
        pltpu.SemaphoreType.DMA((_NB,)),
    ],
)
def _edge_agg(h_hbm, src_hbm, dst_hbm, mask_hbm, ssum_hbm, cnt_hbm,
              acc_sp, cntacc_sp,
              r0, r1, r2, r3, r4, m0, m1, m2, m3, m4,
              s0, s1, s2, s3, s4, d0, d1, d2, d3, d4,
              zrow_v, zcnt_v, gsem, msem, ssem, dsem):
    rows = (r0, r1, r2, r3, r4)
    mvals = (m0, m1, m2, m3, m4)
    sbufs = (s0, s1, s2, s3, s4)
    dbufs = (d0, d1, d2, d3, d4)
    c = lax.axis_index("c")
    s = lax.axis_index("s")
    w = c * _NS + s

    z = jnp.zeros((16,), jnp.float32)
    for i in range(8):
        for j in range(_D // 16):
            zrow_v[i, pl.ds(j * 16, 16)] = z
    _zero_vec16(zcnt_v, _RPS)

    def zb(t, carry):
        pltpu.sync_copy(zrow_v, acc_sp.at[pl.ds(s * _RPS + t * 8, 8)])
        return carry
    lax.fori_loop(0, _RPS // 8, zb, 0)
    pltpu.sync_copy(zcnt_v, cntacc_sp.at[pl.ds(s * _RPS, _RPS)])
    plsc.subcore_barrier()

    base = w * _EW

    def start_idx(j, b):
        pltpu.async_copy(src_hbm.at[pl.ds(base + j * _CH, _CH)],
                         sbufs[b], ssem.at[b])
        pltpu.async_copy(dst_hbm.at[pl.ds(base + j * _CH, _CH)],
                         dbufs[b], dsem.at[b])

    def wait_sidx(j, b):
        pltpu.make_async_copy(src_hbm.at[pl.ds(base + j * _CH, _CH)],
                              sbufs[b], ssem.at[b]).wait()

    def wait_didx(j, b):
        pltpu.make_async_copy(dst_hbm.at[pl.ds(base + j * _CH, _CH)],
                              dbufs[b], dsem.at[b]).wait()

    def start_gather(b):
        pltpu.async_copy(h_hbm.at[sbufs[b]], rows[b], gsem.at[b])
        pltpu.async_copy(mask_hbm.at[sbufs[b]], mvals[b], msem.at[b])

    def wait_gather(b):
        pltpu.make_async_copy(h_hbm.at[sbufs[b]], rows[b],
                              gsem.at[b]).wait()
        pltpu.make_async_copy(mask_hbm.at[sbufs[b]], mvals[b],
                              msem.at[b]).wait()

    for b in range(_NB):
        start_idx(b, b)
    for b in range(_NB - 1):
        wait_sidx(b, b)
        start_gather(b)

    def group(g, carry):
        for b in range(_NB):
            j = g * _NB + b
            wait_gather(b)
            wait_didx(j, b)
            pltpu.sync_copy(rows[b], acc_sp.at[dbufs[b]], add=True)
            pltpu.sync_copy(mvals[b], cntacc_sp.at[dbufs[b]], add=True)

            @pl.when(j + _NB < _NCH)
            def _():
                start_idx(j + _NB, b)

            @pl.when(j + _NB - 1 < _NCH)
            def _():
                bn = (b + _NB - 1) % _NB
                wait_sidx(j + _NB - 1, bn)
                start_gather(bn)
        return carry
    lax.fori_loop(0, _NCH // _NB, group, 0)

    plsc.subcore_barrier()
    pltpu.sync_copy(acc_sp.at[pl.ds(s * _RPS, _RPS)],
                    ssum_hbm.at[c, pl.ds(s * _RPS, _RPS)])
    pltpu.sync_copy(cntacc_sp.at[pl.ds(s * _RPS, _RPS)],
                    cnt_hbm.at[c, pl.ds(s * _RPS, _RPS)])


def _sortkey(bits):
    return jnp.where(bits >= jnp.uint32(0x80000000), ~bits,
                     bits | jnp.uint32(0x80000000))


def _layer_body(ssum_ref, cnt_ref, h_ref, mask_ref, wlt_ref, bl_ref, wrt_ref,
                pw_ref, h_out_ref, mask_out_ref, feat_ref, *, kk):
    ssum = ssum_ref[0] + ssum_ref[1]
    cnt = cnt_ref[0] + cnt_ref[1]
    h = h_ref[...]
    m_prev = mask_ref[...]
    mean = jnp.where(cnt > 0, ssum / jnp.maximum(cnt, 1.0), 0.0)
    out = (jnp.dot(mean, wlt_ref[...], preferred_element_type=jnp.float32)
           + bl_ref[...]
           + jnp.dot(h, wrt_ref[...], preferred_element_type=jnp.float32))
    nrm = jnp.sqrt(jnp.sum(out * out, axis=-1, keepdims=True))
    out = out / jnp.maximum(nrm, 1e-12)
    hraw = jnp.maximum(out, 0.0)
    pw = pw_ref[...]
    score = jnp.tanh(jnp.dot(hraw, pw, preferred_element_type=jnp.float32)
                     / jnp.sqrt(jnp.sum(pw * pw)))
    sel = jnp.where(m_prev > 0, score, -jnp.inf)
    km = _sortkey(lax.bitcast_convert_type(sel, jnp.uint32))
    kf = jnp.float32(kk)

    def body(i, cur):
        bit = lax.shift_left(jnp.uint32(1),
                             jnp.uint32(31) - i.astype(jnp.uint32))
        cand = cur | bit
        n_ge = jnp.sum((km >= cand).astype(jnp.float32))
        return jnp.where(n_ge >= kf, cand, cur)

    th = lax.fori_loop(0, 32, body, jnp.uint32(0))
    m = (km >= th).astype(jnp.float32)
    hnew = hraw * score * m
    h_out_ref[...] = hnew
    mask_out_ref[...] = m
    feat_ref[0:1, :] = jnp.sum(hnew, axis=0, keepdims=True) / jnp.float32(kk)
    feat_ref[1:2, :] = jnp.max(jnp.where(m > 0, hnew, -jnp.inf), axis=0,
                               keepdims=True)


def _layer(ssum_p, cnt_p, h, mask, wlt, bl, wrt, pw, kk):
    return pl.pallas_call(
        functools.partial(_layer_body, kk=kk),
        out_shape=(
            jax.ShapeDtypeStruct((_NP, _D), jnp.float32),
            jax.ShapeDtypeStruct((_NP, 1), jnp.float32),
            jax.ShapeDtypeStruct((2, _D), jnp.float32),
        ),
    )(ssum_p, cnt_p, h, mask, wlt, bl, wrt, pw)


def _head_body(x_ref, inwt_ref, inb_ref, outwt_ref, outb_ref, c1wt_ref,
               c1b_ref, c2wt_ref, c2b_ref, o_ref):
    X = x_ref[...]
    qkv = jnp.dot(X, inwt_ref[...], preferred_element_type=jnp.float32) \
        + inb_ref[...]
    dm, nh, dh = 256, 4, 64
    outs = []
    for hh in range(nh):
        q = qkv[:, hh * dh:(hh + 1) * dh]
        k = qkv[:, dm + hh * dh:dm + (hh + 1) * dh]
        v = qkv[:, 2 * dm + hh * dh:2 * dm + (hh + 1) * dh]
        att = lax.dot_general(q, k, (((1,), (1,)), ((), ())),
                              preferred_element_type=jnp.float32)
        att = jax.nn.softmax(att / jnp.sqrt(jnp.float32(dh)), axis=-1)
        outs.append(jnp.dot(att, v, preferred_element_type=jnp.float32))
    o = jnp.concatenate(outs, axis=1)
    o = jnp.dot(o, outwt_ref[...], preferred_element_type=jnp.float32) \
        + outb_ref[...]
    xm = jnp.mean(o, axis=0, keepdims=True)
    z = jnp.maximum(
        jnp.dot(xm, c1wt_ref[...], preferred_element_type=jnp.float32)
        + c1b_ref[...], 0.0)
    z = jnp.dot(z, c2wt_ref[...], preferred_element_type=jnp.float32) \
        + c2b_ref[...]
    o_ref[...] = jax.nn.sigmoid(z)


def _head(X, inwt, inb, outwt, outb, c1wt, c1b, c2wt, c2b):
    return pl.pallas_call(
        _head_body,
        out_shape=jax.ShapeDtypeStruct((1, 1), jnp.float32),
    )(X, inwt, inb, outwt, outb, c1wt, c1b, c2wt, c2b)


def kernel(x, edge_index, batch, emb, Wl0, bl0, Wr0, pw0, Wl1, bl1, Wr1, pw1,
           Wl2, bl2, Wr2, pw2, in_w, in_b, out_w, out_b, c1w, c1b, c2w, c2b):
    f32 = jnp.float32
    xpad = jnp.concatenate([x[:, 0], jnp.zeros((_NP - _N,), jnp.int32)])
    src = edge_index[0]
    dst = edge_index[1]
    mask = jnp.concatenate([jnp.ones((_N,), f32), jnp.zeros((_NP - _N,), f32)])

    h = _emb_gather(xpad, emb)

    layer_params = ((Wl0, bl0, Wr0, pw0), (Wl1, bl1, Wr1, pw1),
                    (Wl2, bl2, Wr2, pw2))
    feats = []
    for (Wl, bl, Wr, pw), kk in zip(layer_params, _TOPKS):
        ssum_p, cnt_p = _edge_agg(h, src, dst, mask)
        h, mask2, feat = _layer(ssum_p, cnt_p.reshape(_NC, _NP, 1), h,
                                mask.reshape(_NP, 1), Wl.T,
                                bl.reshape(1, _D), Wr.T, pw.reshape(_D, 1),
                                kk)
        mask = mask2.reshape(_NP)
        feats.append(feat.reshape(2 * _D))

    X = jnp.stack(feats, axis=0)
    out = _head(X, in_w.T, in_b.reshape(1, 768), out_w.T,
                out_b.reshape(1, 256), c1w.T, c1b.reshape(1, 128), c2w.T,
                c2b.reshape(1, 1))
    return out.reshape(1)


# speedup vs baseline: 1.3841x; 1.3841x over previous
import functools

import jax
import jax.numpy as jnp
from jax import lax
from jax.experimental import pallas as pl
from jax.experimental.pallas import tpu as pltpu
from jax.experimental.pallas import tpu_sc as plsc

_N = 10000
_NP = 10240
_E = 320000
_D = 128
_TOPKS = (8000, 6400, 5120)

_info = plsc.get_sparse_core_info()
_NC = _info.num_cores
_NS = _info.num_subcores
_NW = _NC * _NS
_EW = _E // _NW
_CH = 40
_NCH = _EW // _CH
_NB = 5
_ND = 2 * _NB
_RPS = _NP // _NS

_mesh = plsc.VectorSubcoreMesh(core_axis_name="c", subcore_axis_name="s")


def _zero_vec16(ref, n):
    z = jnp.zeros((16,), jnp.float32)
    for t in range(n // 16):
        ref[pl.ds(t * 16, 16)] = z


@functools.partial(
    pl.kernel,
    mesh=_mesh,
    out_type=jax.ShapeDtypeStruct((_NP, _D), jnp.float32),
    scratch_types=[
        pltpu.VMEM((_CH,), jnp.int32),
        pltpu.VMEM((_CH, _D), jnp.float32),
        pltpu.SemaphoreType.DMA,
    ],
)
def _emb_gather(xpad_hbm, emb_hbm, out_hbm, idx_v, rows_v, sem):
    c = lax.axis_index("c")
    s = lax.axis_index("s")
    w = c * _NS + s
    rows_per_w = _NP // _NW
    for j in range(rows_per_w // _CH):
        off = w * rows_per_w + j * _CH
        pltpu.sync_copy(xpad_hbm.at[pl.ds(off, _CH)], idx_v)
        pltpu.async_copy(emb_hbm.at[idx_v], rows_v, sem).wait()
        pltpu.sync_copy(rows_v, out_hbm.at[pl.ds(off, _CH)])


@functools.partial(
    pl.kernel,
    mesh=_mesh,
    out_type=[
        jax.ShapeDtypeStruct((_NC, _NP, _D), jnp.float32),
        jax.ShapeDtypeStruct((_NC, _NP), jnp.float32),
    ],
    scratch_types=[
        pltpu.VMEM_SHARED((_NP, _D), jnp.float32),
        pltpu.VMEM_SHARED((_NP,), jnp.float32),
    ] + [pltpu.VMEM((_CH, _D), jnp.float32) for _ in range(_NB)]
      + [pltpu.VMEM((_CH,), jnp.float32) for _ in range(_NB)]
      + [pltpu.VMEM((_CH,), jnp.int32) for _ in range(_NB)]
      + [pltpu.VMEM((_CH,), jnp.int32) for _ in range(_NB)]
      + [
        pltpu.VMEM((8, _D), jnp.float32),
        pltpu.VMEM((_RPS,), jnp.float32),
        pltpu.SemaphoreType.DMA((_NB,)),
        pltpu.SemaphoreType.DMA((_NB,)),
        pltpu.SemaphoreType.DMA((_NB,)),
        pltpu.SemaphoreType.DMA((_NB,)),
    ],
)
def _edge_agg(h_hbm, src_hbm, dst_hbm, mask_hbm, ssum_hbm, cnt_hbm,
              acc_sp, cntacc_sp,
              r0, r1, r2, r3, r4, m0, m1, m2, m3, m4,
              s0, s1, s2, s3, s4, d0, d1, d2, d3, d4,
              zrow_v, zcnt_v, gsem, msem, ssem, dsem):
    rows = (r0, r1, r2, r3, r4)
    mvals = (m0, m1, m2, m3, m4)
    sbufs = (s0, s1, s2, s3, s4)
    dbufs = (d0, d1, d2, d3, d4)
    c = lax.axis_index("c")
    s = lax.axis_index("s")
    w = c * _NS + s

    z = jnp.zeros((16,), jnp.float32)
    for i in range(8):
        for j in range(_D // 16):
            zrow_v[i, pl.ds(j * 16, 16)] = z
    _zero_vec16(zcnt_v, _RPS)

    def zb(t, carry):
        pltpu.sync_copy(zrow_v, acc_sp.at[pl.ds(s * _RPS + t * 8, 8)])
        return carry
    lax.fori_loop(0, _RPS // 8, zb, 0)
    pltpu.sync_copy(zcnt_v, cntacc_sp.at[pl.ds(s * _RPS, _RPS)])
    plsc.subcore_barrier()

    base = w * _EW

    def start_idx(j, b):
        pltpu.async_copy(src_hbm.at[pl.ds(base + j * _CH, _CH)],
                         sbufs[b], ssem.at[b])
        pltpu.async_copy(dst_hbm.at[pl.ds(base + j * _CH, _CH)],
                         dbufs[b], dsem.at[b])

    def wait_sidx(j, b):
        pltpu.make_async_copy(src_hbm.at[pl.ds(base + j * _CH, _CH)],
                              sbufs[b], ssem.at[b]).wait()

    def wait_didx(j, b):
        pltpu.make_async_copy(dst_hbm.at[pl.ds(base + j * _CH, _CH)],
                              dbufs[b], dsem.at[b]).wait()

    def start_gather(b):
        pltpu.async_copy(h_hbm.at[sbufs[b]], rows[b], gsem.at[b])
        pltpu.async_copy(mask_hbm.at[sbufs[b]], mvals[b], msem.at[b])

    def wait_gather(b):
        pltpu.make_async_copy(h_hbm.at[sbufs[b]], rows[b],
                              gsem.at[b]).wait()
        pltpu.make_async_copy(mask_hbm.at[sbufs[b]], mvals[b],
                              msem.at[b]).wait()

    for b in range(_NB):
        start_idx(b, b)
    for b in range(_NB - 1):
        wait_sidx(b, b)
        start_gather(b)

    def group(g, carry):
        for b in range(_NB):
            j = g * _NB + b
            wait_gather(b)
            wait_didx(j, b)
            pltpu.sync_copy(rows[b], acc_sp.at[dbufs[b]], add=True)
            pltpu.sync_copy(mvals[b], cntacc_sp.at[dbufs[b]], add=True)

            @pl.when(j + _NB < _NCH)
            def _():
                start_idx(j + _NB, b)

            @pl.when(j + _NB - 1 < _NCH)
            def _():
                bn = (b + _NB - 1) % _NB
                wait_sidx(j + _NB - 1, bn)
                start_gather(bn)
        return carry
    lax.fori_loop(0, _NCH // _NB, group, 0)

    plsc.subcore_barrier()
    pltpu.sync_copy(acc_sp.at[pl.ds(s * _RPS, _RPS)],
                    ssum_hbm.at[c, pl.ds(s * _RPS, _RPS)])
    pltpu.sync_copy(cntacc_sp.at[pl.ds(s * _RPS, _RPS)],
                    cnt_hbm.at[c, pl.ds(s * _RPS, _RPS)])


def _sortkey(bits):
    return jnp.where(bits >= jnp.uint32(0x80000000), ~bits,
                     bits | jnp.uint32(0x80000000))


_NR = _NP // _D


def _layer_body(ssum_ref, cnt_ref, h_ref, mask_ref, wlt_ref, bl_ref, wrt_ref,
                pw_ref, h_out_ref, mask_out_ref, feat_ref, *, kk):
    ssum3 = (ssum_ref[0] + ssum_ref[1]).reshape(_NR, _D, _D)
    cnt = cnt_ref[0] + cnt_ref[1]
    h = h_ref[...]
    m_prev = mask_ref[...]
    cnt3 = cnt[:, :, None]
    mean = jnp.where(cnt3 > 0, ssum3 / jnp.maximum(cnt3, 1.0),
                     0.0).reshape(_NP, _D)
    out = (jnp.dot(mean, wlt_ref[...], preferred_element_type=jnp.float32)
           + bl_ref[...]
           + jnp.dot(h, wrt_ref[...], preferred_element_type=jnp.float32))
    nrm = jnp.sqrt(jnp.sum(out * out, axis=-1, keepdims=True))
    out = out / jnp.maximum(nrm, 1e-12)
    hraw = jnp.maximum(out, 0.0)
    hraw3 = hraw.reshape(_NR, _D, _D)
    pw = pw_ref[...]
    score = jnp.tanh(jnp.sum(hraw3 * pw[0], axis=-1)
                     / jnp.sqrt(jnp.sum(pw * pw)))
    sel = jnp.where(m_prev > 0, score, -jnp.inf)
    km = _sortkey(lax.bitcast_convert_type(sel, jnp.uint32))
    kf = jnp.float32(kk)

    def body(i, cur):
        bit = lax.shift_left(jnp.uint32(1),
                             jnp.uint32(31) - i.astype(jnp.uint32))
        cand = cur | bit
        n_ge = jnp.sum((km >= cand).astype(jnp.float32))
        return jnp.where(n_ge >= kf, cand, cur)

    th = lax.fori_loop(0, 32, body, jnp.uint32(0))
    m = (km >= th).astype(jnp.float32)
    hnew3 = hraw3 * (score * m)[:, :, None]
    h_out_ref[...] = hnew3.reshape(_NP, _D)
    mask_out_ref[...] = m
    feat_ref[0:1, :] = (jnp.sum(jnp.sum(hnew3, axis=0), axis=0,
                                keepdims=True) / jnp.float32(kk))
    hmasked = jnp.where(m[:, :, None] > 0, hnew3, -jnp.inf)
    feat_ref[1:2, :] = jnp.max(jnp.max(hmasked, axis=0), axis=0,
                               keepdims=True)


def _layer(ssum_p, cnt_p, h, mask, wlt, bl, wrt, pw, kk):
    return pl.pallas_call(
        functools.partial(_layer_body, kk=kk),
        out_shape=(
            jax.ShapeDtypeStruct((_NP, _D), jnp.float32),
            jax.ShapeDtypeStruct((_NR, _D), jnp.float32),
            jax.ShapeDtypeStruct((2, _D), jnp.float32),
        ),
    )(ssum_p, cnt_p, h, mask, wlt, bl, wrt, pw)


def _head_body(x_ref, inwt_ref, inb_ref, outwt_ref, outb_ref, c1wt_ref,
               c1b_ref, c2wt_ref, c2b_ref, o_ref):
    X = x_ref[...]
    qkv = jnp.dot(X, inwt_ref[...], preferred_element_type=jnp.float32) \
        + inb_ref[...]
    dm, nh, dh = 256, 4, 64
    outs = []
    for hh in range(nh):
        q = qkv[:, hh * dh:(hh + 1) * dh]
        k = qkv[:, dm + hh * dh:dm + (hh + 1) * dh]
        v = qkv[:, 2 * dm + hh * dh:2 * dm + (hh + 1) * dh]
        att = lax.dot_general(q, k, (((1,), (1,)), ((), ())),
                              preferred_element_type=jnp.float32)
        att = jax.nn.softmax(att / jnp.sqrt(jnp.float32(dh)), axis=-1)
        outs.append(jnp.dot(att, v, preferred_element_type=jnp.float32))
    o = jnp.concatenate(outs, axis=1)
    o = jnp.dot(o, outwt_ref[...], preferred_element_type=jnp.float32) \
        + outb_ref[...]
    xm = jnp.mean(o, axis=0, keepdims=True)
    z = jnp.maximum(
        jnp.dot(xm, c1wt_ref[...], preferred_element_type=jnp.float32)
        + c1b_ref[...], 0.0)
    z = jnp.dot(z, c2wt_ref[...], preferred_element_type=jnp.float32) \
        + c2b_ref[...]
    o_ref[...] = jax.nn.sigmoid(z)


def _head(X, inwt, inb, outwt, outb, c1wt, c1b, c2wt, c2b):
    return pl.pallas_call(
        _head_body,
        out_shape=jax.ShapeDtypeStruct((1, 1), jnp.float32),
    )(X, inwt, inb, outwt, outb, c1wt, c1b, c2wt, c2b)


def kernel(x, edge_index, batch, emb, Wl0, bl0, Wr0, pw0, Wl1, bl1, Wr1, pw1,
           Wl2, bl2, Wr2, pw2, in_w, in_b, out_w, out_b, c1w, c1b, c2w, c2b):
    f32 = jnp.float32
    xpad = jnp.concatenate([x[:, 0], jnp.zeros((_NP - _N,), jnp.int32)])
    src = edge_index[0]
    dst = edge_index[1]
    mask = jnp.concatenate([jnp.ones((_N,), f32), jnp.zeros((_NP - _N,), f32)])

    h = _emb_gather(xpad, emb)

    layer_params = ((Wl0, bl0, Wr0, pw0), (Wl1, bl1, Wr1, pw1),
                    (Wl2, bl2, Wr2, pw2))
    feats = []
    for (Wl, bl, Wr, pw), kk in zip(layer_params, _TOPKS):
        ssum_p, cnt_p = _edge_agg(h, src, dst, mask)
        h, mask2, feat = _layer(ssum_p, cnt_p.reshape(_NC, _NR, _D), h,
                                mask.reshape(_NR, _D), Wl.T,
                                bl.reshape(1, _D), Wr.T, pw.reshape(1, _D),
                                kk)
        mask = mask2.reshape(_NP)
        feats.append(feat.reshape(2 * _D))

    X = jnp.stack(feats, axis=0)
    out = _head(X, in_w.T, in_b.reshape(1, 768), out_w.T,
                out_b.reshape(1, 256), c1w.T, c1b.reshape(1, 128), c2w.T,
                c2b.reshape(1, 1))
    return out.reshape(1)


# speedup vs baseline: 1.4206x; 1.0264x over previous
import functools

import jax
import jax.numpy as jnp
from jax import lax
from jax.experimental import pallas as pl
from jax.experimental.pallas import tpu as pltpu
from jax.experimental.pallas import tpu_sc as plsc

_N = 10000
_NP = 10240
_E = 320000
_D = 128
_TOPKS = (8000, 6400, 5120)

_info = plsc.get_sparse_core_info()
_NC = _info.num_cores
_NS = _info.num_subcores
_NW = _NC * _NS
_EW = _E // _NW
_CH = 40
_NCH = _EW // _CH
_NB = 5
_ND = 2 * _NB
_RPS = _NP // _NS

_mesh = plsc.VectorSubcoreMesh(core_axis_name="c", subcore_axis_name="s")


def _zero_vec16(ref, n):
    z = jnp.zeros((16,), jnp.float32)
    for t in range(n // 16):
        ref[pl.ds(t * 16, 16)] = z


@functools.partial(
    pl.kernel,
    mesh=_mesh,
    out_type=jax.ShapeDtypeStruct((_NP, _D), jnp.float32),
    scratch_types=[
        pltpu.VMEM((_CH,), jnp.int32),
        pltpu.VMEM((_CH, _D), jnp.float32),
        pltpu.SemaphoreType.DMA,
    ],
)
def _emb_gather(xpad_hbm, emb_hbm, out_hbm, idx_v, rows_v, sem):
    c = lax.axis_index("c")
    s = lax.axis_index("s")
    w = c * _NS + s
    rows_per_w = _NP // _NW
    for j in range(rows_per_w // _CH):
        off = w * rows_per_w + j * _CH
        pltpu.sync_copy(xpad_hbm.at[pl.ds(off, _CH)], idx_v)
        pltpu.async_copy(emb_hbm.at[idx_v], rows_v, sem).wait()
        pltpu.sync_copy(rows_v, out_hbm.at[pl.ds(off, _CH)])


@functools.partial(
    pl.kernel,
    mesh=_mesh,
    out_type=[
        jax.ShapeDtypeStruct((_NC, _NP, _D), jnp.float32),
        jax.ShapeDtypeStruct((_NC, _NP), jnp.float32),
    ],
    scratch_types=[
        pltpu.VMEM_SHARED((_NP, _D), jnp.float32),
        pltpu.VMEM_SHARED((_NP,), jnp.float32),
    ] + [pltpu.VMEM((_CH, _D), jnp.float32) for _ in range(_NB)]
      + [pltpu.VMEM((_CH,), jnp.float32) for _ in range(_NB)]
      + [pltpu.VMEM((_CH,), jnp.int32) for _ in range(_NB)]
      + [pltpu.VMEM((_CH,), jnp.int32) for _ in range(_NB)]
      + [
        pltpu.VMEM((8, _D), jnp.float32),
        pltpu.VMEM((_RPS,), jnp.float32),
        pltpu.SemaphoreType.DMA((_NB,)),
        pltpu.SemaphoreType.DMA((_NB,)),
        pltpu.SemaphoreType.DMA((_NB,)),
        pltpu.SemaphoreType.DMA((_NB,)),
        pltpu.SemaphoreType.DMA,
    ],
)
def _edge_agg(h_hbm, src_hbm, dst_hbm, mask_hbm, ssum_hbm, cnt_hbm,
              acc_sp, cntacc_sp,
              r0, r1, r2, r3, r4, m0, m1, m2, m3, m4,
              s0, s1, s2, s3, s4, d0, d1, d2, d3, d4,
              zrow_v, zcnt_v, gsem, msem, ssem, dsem, csem):
    rows = (r0, r1, r2, r3, r4)
    mvals = (m0, m1, m2, m3, m4)
    sbufs = (s0, s1, s2, s3, s4)
    dbufs = (d0, d1, d2, d3, d4)
    c = lax.axis_index("c")
    s = lax.axis_index("s")
    w = c * _NS + s

    z = jnp.zeros((16,), jnp.float32)
    for i in range(8):
        for j in range(_D // 16):
            zrow_v[i, pl.ds(j * 16, 16)] = z
    _zero_vec16(zcnt_v, _RPS)

    def zb(t, carry):
        pltpu.sync_copy(zrow_v, acc_sp.at[pl.ds(s * _RPS + t * 8, 8)])
        return carry
    lax.fori_loop(0, _RPS // 8, zb, 0)
    pltpu.sync_copy(zcnt_v, cntacc_sp.at[pl.ds(s * _RPS, _RPS)])
    plsc.subcore_barrier()

    base = w * _EW

    def start_idx(j, b):
        pltpu.async_copy(src_hbm.at[pl.ds(base + j * _CH, _CH)],
                         sbufs[b], ssem.at[b])
        pltpu.async_copy(dst_hbm.at[pl.ds(base + j * _CH, _CH)],
                         dbufs[b], dsem.at[b])

    def wait_sidx(j, b):
        pltpu.make_async_copy(src_hbm.at[pl.ds(base + j * _CH, _CH)],
                              sbufs[b], ssem.at[b]).wait()

    def wait_didx(j, b):
        pltpu.make_async_copy(dst_hbm.at[pl.ds(base + j * _CH, _CH)],
                              dbufs[b], dsem.at[b]).wait()

    def start_gather(b):
        pltpu.async_copy(h_hbm.at[sbufs[b]], rows[b], gsem.at[b])
        pltpu.async_copy(mask_hbm.at[sbufs[b]], mvals[b], msem.at[b])

    def wait_gather(b):
        pltpu.make_async_copy(h_hbm.at[sbufs[b]], rows[b],
                              gsem.at[b]).wait()
        pltpu.make_async_copy(mask_hbm.at[sbufs[b]], mvals[b],
                              msem.at[b]).wait()

    for b in range(_NB):
        start_idx(b, b)
    for b in range(_NB - 1):
        wait_sidx(b, b)
        start_gather(b)

    def group(g, carry):
        for b in range(_NB):
            j = g * _NB + b
            wait_gather(b)
            wait_didx(j, b)
            cnt_copy = pltpu.async_copy(mvals[b], cntacc_sp.at[dbufs[b]],
                                        csem, add=True)
            pltpu.sync_copy(rows[b], acc_sp.at[dbufs[b]], add=True)
            cnt_copy.wait()

            @pl.when(j + _NB < _NCH)
            def _():
                start_idx(j + _NB, b)

            @pl.when(j + _NB - 1 < _NCH)
            def _():
                bn = (b + _NB - 1) % _NB
                wait_sidx(j + _NB - 1, bn)
                start_gather(bn)
        return carry
    lax.fori_loop(0, _NCH // _NB, group, 0)

    plsc.subcore_barrier()
    pltpu.sync_copy(acc_sp.at[pl.ds(s * _RPS, _RPS)],
                    ssum_hbm.at[c, pl.ds(s * _RPS, _RPS)])
    pltpu.sync_copy(cntacc_sp.at[pl.ds(s * _RPS, _RPS)],
                    cnt_hbm.at[c, pl.ds(s * _RPS, _RPS)])


def _sortkey(bits):
    return jnp.where(bits >= jnp.uint32(0x80000000), ~bits,
                     bits | jnp.uint32(0x80000000))


_NR = _NP // _D


def _layer_body(ssum_ref, cnt_ref, h_ref, mask_ref, wlt_ref, bl_ref, wrt_ref,
                pw_ref, h_out_ref, mask_out_ref, feat_ref, *, kk):
    ssum3 = (ssum_ref[0] + ssum_ref[1]).reshape(_NR, _D, _D)
    cnt = cnt_ref[0] + cnt_ref[1]
    h = h_ref[...]
    m_prev = mask_ref[...]
    cnt3 = cnt[:, :, None]
    mean = jnp.where(cnt3 > 0, ssum3 / jnp.maximum(cnt3, 1.0),
                     0.0).reshape(_NP, _D)
    out = (jnp.dot(mean, wlt_ref[...], preferred_element_type=jnp.float32)
           + bl_ref[...]
           + jnp.dot(h, wrt_ref[...], preferred_element_type=jnp.float32))
    nrm = jnp.sqrt(jnp.sum(out * out, axis=-1, keepdims=True))
    out = out / jnp.maximum(nrm, 1e-12)
    hraw = jnp.maximum(out, 0.0)
    hraw3 = hraw.reshape(_NR, _D, _D)
    pw = pw_ref[...]
    score = jnp.tanh(jnp.sum(hraw3 * pw[0], axis=-1)
                     / jnp.sqrt(jnp.sum(pw * pw)))
    sel = jnp.where(m_prev > 0, score, -jnp.inf)
    km = _sortkey(lax.bitcast_convert_type(sel, jnp.uint32))
    kf = jnp.float32(kk)

    def body(i, cur):
        bit = lax.shift_left(jnp.uint32(1),
                             jnp.uint32(31) - i.astype(jnp.uint32))
        cand = cur | bit
        n_ge = jnp.sum((km >= cand).astype(jnp.float32))
        return jnp.where(n_ge >= kf, cand, cur)

    th = lax.fori_loop(0, 32, body, jnp.uint32(0))
    m = (km >= th).astype(jnp.float32)
    hnew3 = hraw3 * (score * m)[:, :, None]
    h_out_ref[...] = hnew3.reshape(_NP, _D)
    mask_out_ref[...] = m
    feat_ref[0:1, :] = (jnp.sum(jnp.sum(hnew3, axis=0), axis=0,
                                keepdims=True) / jnp.float32(kk))
    hmasked = jnp.where(m[:, :, None] > 0, hnew3, -jnp.inf)
    feat_ref[1:2, :] = jnp.max(jnp.max(hmasked, axis=0), axis=0,
                               keepdims=True)


def _layer(ssum_p, cnt_p, h, mask, wlt, bl, wrt, pw, kk):
    return pl.pallas_call(
        functools.partial(_layer_body, kk=kk),
        out_shape=(
            jax.ShapeDtypeStruct((_NP, _D), jnp.float32),
            jax.ShapeDtypeStruct((_NR, _D), jnp.float32),
            jax.ShapeDtypeStruct((2, _D), jnp.float32),
        ),
    )(ssum_p, cnt_p, h, mask, wlt, bl, wrt, pw)


def _head_body(x_ref, inwt_ref, inb_ref, outwt_ref, outb_ref, c1wt_ref,
               c1b_ref, c2wt_ref, c2b_ref, o_ref):
    X = x_ref[...]
    qkv = jnp.dot(X, inwt_ref[...], preferred_element_type=jnp.float32) \
        + inb_ref[...]
    dm, nh, dh = 256, 4, 64
    outs = []
    for hh in range(nh):
        q = qkv[:, hh * dh:(hh + 1) * dh]
        k = qkv[:, dm + hh * dh:dm + (hh + 1) * dh]
        v = qkv[:, 2 * dm + hh * dh:2 * dm + (hh + 1) * dh]
        att = lax.dot_general(q, k, (((1,), (1,)), ((), ())),
                              preferred_element_type=jnp.float32)
        att = jax.nn.softmax(att / jnp.sqrt(jnp.float32(dh)), axis=-1)
        outs.append(jnp.dot(att, v, preferred_element_type=jnp.float32))
    o = jnp.concatenate(outs, axis=1)
    o = jnp.dot(o, outwt_ref[...], preferred_element_type=jnp.float32) \
        + outb_ref[...]
    xm = jnp.mean(o, axis=0, keepdims=True)
    z = jnp.maximum(
        jnp.dot(xm, c1wt_ref[...], preferred_element_type=jnp.float32)
        + c1b_ref[...], 0.0)
    z = jnp.dot(z, c2wt_ref[...], preferred_element_type=jnp.float32) \
        + c2b_ref[...]
    o_ref[...] = jax.nn.sigmoid(z)


def _head(X, inwt, inb, outwt, outb, c1wt, c1b, c2wt, c2b):
    return pl.pallas_call(
        _head_body,
        out_shape=jax.ShapeDtypeStruct((1, 1), jnp.float32),
    )(X, inwt, inb, outwt, outb, c1wt, c1b, c2wt, c2b)


def kernel(x, edge_index, batch, emb, Wl0, bl0, Wr0, pw0, Wl1, bl1, Wr1, pw1,
           Wl2, bl2, Wr2, pw2, in_w, in_b, out_w, out_b, c1w, c1b, c2w, c2b):
    f32 = jnp.float32
    xpad = jnp.concatenate([x[:, 0], jnp.zeros((_NP - _N,), jnp.int32)])
    src = edge_index[0]
    dst = edge_index[1]
    mask = jnp.concatenate([jnp.ones((_N,), f32), jnp.zeros((_NP - _N,), f32)])

    h = _emb_gather(xpad, emb)

    layer_params = ((Wl0, bl0, Wr0, pw0), (Wl1, bl1, Wr1, pw1),
                    (Wl2, bl2, Wr2, pw2))
    feats = []
    for (Wl, bl, Wr, pw), kk in zip(layer_params, _TOPKS):
        ssum_p, cnt_p = _edge_agg(h, src, dst, mask)
        h, mask2, feat = _layer(ssum_p, cnt_p.reshape(_NC, _NR, _D), h,
                                mask.reshape(_NR, _D), Wl.T,
                                bl.reshape(1, _D), Wr.T, pw.reshape(1, _D),
                                kk)
        mask = mask2.reshape(_NP)
        feats.append(feat.reshape(2 * _D))

    X = jnp.stack(feats, axis=0)
    out = _head(X, in_w.T, in_b.reshape(1, 768), out_w.T,
                out_b.reshape(1, 256), c1w.T, c1b.reshape(1, 128), c2w.T,
                c2b.reshape(1, 1))
    return out.reshape(1)
